# Initial kernel scaffold; baseline (speedup 1.0000x reference)
import jax
import jax.numpy as jnp
from jax.experimental import pallas as pl


def kernel(logits0, logits1, logits2, logits3, logits4, regress0, regress1, regress2, regress3, regress4, anchors):
    raise NotImplementedError("write your pallas kernel here")


# speedup vs baseline: 1.0349x; 1.0349x over previous
import functools

import jax
import jax.numpy as jnp
from jax.experimental import pallas as pl

_B = 8
_C = 80
_A = 9
_SP = (4096, 1024, 256, 64, 16)
_K = 1000
_PAD = 1024
_THR = 0.05
_OUT = 100
_IOU = 0.5


def _nms_body(in_ref, out_ref):
    x = in_ref[0]
    ax1, ay1, ax2, ay2 = x[0], x[1], x[2], x[3]
    d0, d1, d2, d3 = x[4], x[5], x[6], x[7]
    logit = x[8]
    lab = x[9]

    wa = ax2 - ax1
    ha = ay2 - ay1
    cxa = ax1 + 0.5 * wa
    cya = ay1 + 0.5 * ha
    cx = d0 * wa + cxa
    cy = d1 * ha + cya
    w = wa * jnp.exp(d2)
    h = ha * jnp.exp(d3)
    bx1 = cx - 0.5 * w
    by1 = cy - 0.5 * h
    bx2 = cx + 0.5 * w
    by2 = cy + 0.5 * h

    s = 1.0 / (1.0 + jnp.exp(-logit))

    off = lab * 4096.0
    ox1 = bx1 + off
    oy1 = by1 + off
    ox2 = bx2 + off
    oy2 = by2 + off
    areas = jnp.maximum(ox2 - ox1, 0.0) * jnp.maximum(oy2 - oy1, 0.0)

    cur0 = jnp.where(s > _THR, s, -1.0)

    idx = (jax.lax.broadcasted_iota(jnp.int32, (8, 128), 0) * 128
           + jax.lax.broadcasted_iota(jnp.int32, (8, 128), 1))
    lane = jax.lax.broadcasted_iota(jnp.int32, (1, 128), 1)
    rowid = jax.lax.broadcasted_iota(jnp.int32, (_OUT, 128), 0)
    rows0 = jnp.zeros((_OUT, 128), jnp.float32)

    def body(t, carry):
        cur, rows = carry
        m = jnp.max(cur)
        keep = m > 0.0
        sel = jnp.min(jnp.where(cur == m, idx, jnp.int32(1 << 30)))
        sm = idx == sel

        def pick(v):
            return jnp.sum(jnp.where(sm, v, 0.0))

        px1 = pick(bx1)
        py1 = pick(by1)
        px2 = pick(bx2)
        py2 = pick(by2)
        plab = pick(lab)
        qx1 = pick(ox1)
        qy1 = pick(oy1)
        qx2 = pick(ox2)
        qy2 = pick(oy2)
        parea = pick(areas)

        inter = (jnp.maximum(jnp.minimum(qx2, ox2) - jnp.maximum(qx1, ox1), 0.0)
                 * jnp.maximum(jnp.minimum(qy2, oy2) - jnp.maximum(qy1, oy1), 0.0))
        iou = inter / (parea + areas - inter + 1e-9)
        cur = jnp.where(iou > _IOU, -1.0, cur)

        keepf = jnp.where(keep, 1.0, 0.0)
        rowv = (jnp.where(lane == 0, px1, 0.0)
                + jnp.where(lane == 1, py1, 0.0)
                + jnp.where(lane == 2, px2, 0.0)
                + jnp.where(lane == 3, py2, 0.0)
                + jnp.where(lane == 4, m, 0.0)
                + jnp.where(lane == 5, plab, 0.0)) * keepf
        rows = jnp.where(rowid == t, rowv, rows)
        return cur, rows

    _, rows = jax.lax.fori_loop(0, _OUT, body, (cur0, rows0))
    out_ref[0] = rows


@jax.jit
def _impl(logits, regress, anchors):
    cand_s, cand_a, cand_c, cand_d = [], [], [], []
    off = 0
    for lv, s in enumerate(_SP):
        lg = logits[lv].reshape(_B, _A * _C * s)
        sc, j = jax.lax.top_k(lg, _K)
        pos = j % s
        ch = j // s
        k = ch // _C
        c = ch % _C
        a = off + pos * _A + k
        rg = regress[lv].reshape(_B, _A * 4 * s)
        base = (k * 4) * s + pos
        d = [jnp.take_along_axis(rg, base + t * s, axis=1) for t in range(4)]
        cand_s.append(sc)
        cand_a.append(a)
        cand_c.append(c)
        cand_d.append(jnp.stack(d, axis=-1))
        off += s * _A

    S = jnp.concatenate(cand_s, axis=1)
    A = jnp.concatenate(cand_a, axis=1)
    Cc = jnp.concatenate(cand_c, axis=1)
    D = jnp.concatenate(cand_d, axis=1)

    gidx = A * _C + Cc
    srt = jax.lax.sort(
        (-S, gidx, A, Cc, D[..., 0], D[..., 1], D[..., 2], D[..., 3]),
        dimension=1, num_keys=2)
    ms = -srt[0][:, :_K]
    a_sel = srt[2][:, :_K]
    c_sel = srt[3][:, :_K]
    d_sel = jnp.stack([srt[4][:, :_K], srt[5][:, :_K],
                       srt[6][:, :_K], srt[7][:, :_K]], axis=-1)
    anc = anchors[a_sel]

    npad = _PAD - _K

    def p(v, fill):
        return jnp.pad(v, ((0, 0), (0, npad)), constant_values=fill)

    packed = jnp.stack(
        [p(anc[..., 0], 0.0), p(anc[..., 1], 0.0),
         p(anc[..., 2], 0.0), p(anc[..., 3], 0.0),
         p(d_sel[..., 0], 0.0), p(d_sel[..., 1], 0.0),
         p(d_sel[..., 2], 0.0), p(d_sel[..., 3], 0.0),
         p(ms, -1e9), p(c_sel.astype(jnp.float32), 0.0)],
        axis=1).reshape(_B, 10, 8, 128)

    out = pl.pallas_call(
        _nms_body,
        grid=(_B,),
        in_specs=[pl.BlockSpec((1, 10, 8, 128), lambda b: (b, 0, 0, 0))],
        out_specs=pl.BlockSpec((1, _OUT, 128), lambda b: (b, 0, 0)),
        out_shape=jax.ShapeDtypeStruct((_B, _OUT, 128), jnp.float32),
    )(packed)
    return out[:, :, :6]


def kernel(logits0, logits1, logits2, logits3, logits4,
           regress0, regress1, regress2, regress3, regress4, anchors):
    return _impl([logits0, logits1, logits2, logits3, logits4],
                 [regress0, regress1, regress2, regress3, regress4], anchors)


# speedup vs baseline: 4.5212x; 4.3687x over previous
import functools

import jax
import jax.numpy as jnp
from jax.experimental import pallas as pl

_B = 8
_C = 80
_A = 9
_SP = (4096, 1024, 256, 64, 16)
_K = 1000
_PAD = 1024
_THR = 0.05
_OUT = 100
_IOU = 0.5

_N = sum(_A * _C * s for s in _SP)
_CS = 512
_CHUNKS = 7680
_NP = _CHUNKS * _CS
_KC = 1152


def _chunkmax_body(in_ref, out_ref):
    x = in_ref[0]
    m = jnp.max(x, axis=1)
    out_ref[0] = jnp.broadcast_to(m[:, None], (_CHUNKS, 8))


def _nms_body(in_ref, out_ref):
    x = in_ref[0]
    ax1, ay1, ax2, ay2 = x[0], x[1], x[2], x[3]
    d0, d1, d2, d3 = x[4], x[5], x[6], x[7]
    logit = x[8]
    lab = x[9]

    wa = ax2 - ax1
    ha = ay2 - ay1
    cxa = ax1 + 0.5 * wa
    cya = ay1 + 0.5 * ha
    cx = d0 * wa + cxa
    cy = d1 * ha + cya
    w = wa * jnp.exp(d2)
    h = ha * jnp.exp(d3)
    bx1 = cx - 0.5 * w
    by1 = cy - 0.5 * h
    bx2 = cx + 0.5 * w
    by2 = cy + 0.5 * h

    s = 1.0 / (1.0 + jnp.exp(-logit))

    off = lab * 4096.0
    ox1 = bx1 + off
    oy1 = by1 + off
    ox2 = bx2 + off
    oy2 = by2 + off
    areas = jnp.maximum(ox2 - ox1, 0.0) * jnp.maximum(oy2 - oy1, 0.0)

    cur0 = jnp.where(s > _THR, s, -1.0)

    idx = (jax.lax.broadcasted_iota(jnp.int32, (8, 128), 0) * 128
           + jax.lax.broadcasted_iota(jnp.int32, (8, 128), 1))
    lane = jax.lax.broadcasted_iota(jnp.int32, (1, 128), 1)
    rowid = jax.lax.broadcasted_iota(jnp.int32, (_OUT, 128), 0)
    rows0 = jnp.zeros((_OUT, 128), jnp.float32)

    def body(t, carry):
        cur, rows = carry
        m = jnp.max(cur)
        keep = m > 0.0
        sel = jnp.min(jnp.where(cur == m, idx, jnp.int32(1 << 30)))
        sm = idx == sel

        def pick(v):
            return jnp.sum(jnp.where(sm, v, 0.0))

        px1 = pick(bx1)
        py1 = pick(by1)
        px2 = pick(bx2)
        py2 = pick(by2)
        plab = pick(lab)
        qx1 = pick(ox1)
        qy1 = pick(oy1)
        qx2 = pick(ox2)
        qy2 = pick(oy2)
        parea = pick(areas)

        inter = (jnp.maximum(jnp.minimum(qx2, ox2) - jnp.maximum(qx1, ox1), 0.0)
                 * jnp.maximum(jnp.minimum(qy2, oy2) - jnp.maximum(qy1, oy1), 0.0))
        iou = inter / (parea + areas - inter + 1e-9)
        cur = jnp.where(iou > _IOU, -1.0, cur)

        keepf = jnp.where(keep, 1.0, 0.0)
        rowv = (jnp.where(lane == 0, px1, 0.0)
                + jnp.where(lane == 1, py1, 0.0)
                + jnp.where(lane == 2, px2, 0.0)
                + jnp.where(lane == 3, py2, 0.0)
                + jnp.where(lane == 4, m, 0.0)
                + jnp.where(lane == 5, plab, 0.0)) * keepf
        rows = jnp.where(rowid == t, rowv, rows)
        return cur, rows

    _, rows = jax.lax.fori_loop(0, _OUT, body, (cur0, rows0))
    out_ref[0] = rows


@jax.jit
def _impl(logits, regress, anchors):
    flat = jnp.concatenate([logits[lv].reshape(_B, -1) for lv in range(5)],
                           axis=1)
    flat = jnp.pad(flat, ((0, 0), (0, _NP - _N)), constant_values=-1e30)
    fx = flat.reshape(_B, _CHUNKS, _CS)

    cmax = pl.pallas_call(
        _chunkmax_body,
        grid=(_B,),
        in_specs=[pl.BlockSpec((1, _CHUNKS, _CS), lambda b: (b, 0, 0))],
        out_specs=pl.BlockSpec((1, _CHUNKS, 8), lambda b: (b, 0, 0)),
        out_shape=jax.ShapeDtypeStruct((_B, _CHUNKS, 8), jnp.float32),
    )(fx)[:, :, 0]

    _, cid = jax.lax.top_k(cmax, _KC)
    cand = jnp.take_along_axis(fx, cid[:, :, None], axis=1)
    sc, p = jax.lax.top_k(cand.reshape(_B, _KC * _CS), _KC)
    cj = jnp.take_along_axis(cid, p // _CS, axis=1)
    j = cj * _CS + p % _CS

    zero = jnp.zeros_like(j)
    a = zero
    c = zero
    r0 = zero
    rs = zero
    base = 0
    off_a = 0
    off_r = 0
    for s in _SP:
        m = (j >= base) & (j < base + _A * _C * s)
        local = j - base
        ch = local // s
        pos = local % s
        k = ch // _C
        cc = ch % _C
        a = jnp.where(m, off_a + pos * _A + k, a)
        c = jnp.where(m, cc, c)
        r0 = jnp.where(m, off_r + (k * 4) * s + pos, r0)
        rs = jnp.where(m, s, rs)
        base += _A * _C * s
        off_a += _A * s
        off_r += _A * 4 * s

    rflat = jnp.concatenate([regress[lv].reshape(_B, -1) for lv in range(5)],
                            axis=1)
    d = [jnp.take_along_axis(rflat, r0 + t * rs, axis=1) for t in range(4)]

    gidx = a * _C + c
    srt = jax.lax.sort(
        (-sc, gidx, a, c, d[0], d[1], d[2], d[3]),
        dimension=1, num_keys=2)
    ms = -srt[0][:, :_K]
    a_sel = srt[2][:, :_K]
    c_sel = srt[3][:, :_K]
    d_sel = jnp.stack([srt[4][:, :_K], srt[5][:, :_K],
                       srt[6][:, :_K], srt[7][:, :_K]], axis=-1)
    anc = anchors[a_sel]

    npad = _PAD - _K

    def p(v, fill):
        return jnp.pad(v, ((0, 0), (0, npad)), constant_values=fill)

    packed = jnp.stack(
        [p(anc[..., 0], 0.0), p(anc[..., 1], 0.0),
         p(anc[..., 2], 0.0), p(anc[..., 3], 0.0),
         p(d_sel[..., 0], 0.0), p(d_sel[..., 1], 0.0),
         p(d_sel[..., 2], 0.0), p(d_sel[..., 3], 0.0),
         p(ms, -1e9), p(c_sel.astype(jnp.float32), 0.0)],
        axis=1).reshape(_B, 10, 8, 128)

    out = pl.pallas_call(
        _nms_body,
        grid=(_B,),
        in_specs=[pl.BlockSpec((1, 10, 8, 128), lambda b: (b, 0, 0, 0))],
        out_specs=pl.BlockSpec((1, _OUT, 128), lambda b: (b, 0, 0)),
        out_shape=jax.ShapeDtypeStruct((_B, _OUT, 128), jnp.float32),
    )(packed)
    return out[:, :, :6]


def kernel(logits0, logits1, logits2, logits3, logits4,
           regress0, regress1, regress2, regress3, regress4, anchors):
    return _impl([logits0, logits1, logits2, logits3, logits4],
                 [regress0, regress1, regress2, regress3, regress4], anchors)


# speedup vs baseline: 6.1366x; 1.3573x over previous
import functools

import jax
import jax.numpy as jnp
from jax.experimental import pallas as pl

_B = 8
_C = 80
_A = 9
_SP = (4096, 1024, 256, 64, 16)
_K = 1000
_PAD = 1024
_THR = 0.05
_OUT = 100
_IOU = 0.5

_N = sum(_A * _C * s for s in _SP)
_CS = 128
_CHUNKS = 30720
_NP = _CHUNKS * _CS
_KC = 1152


_CBLK = 3840


def _chunkmax_body(in_ref, out_ref):
    x = in_ref[0]
    m = jnp.max(x, axis=1)
    out_ref[0] = jnp.broadcast_to(m[:, None], (_CBLK, 8))


def _nms_body(in_ref, out_ref):
    x = in_ref[0]
    ax1, ay1, ax2, ay2 = x[0], x[1], x[2], x[3]
    d0, d1, d2, d3 = x[4], x[5], x[6], x[7]
    logit = x[8]
    lab = x[9]

    wa = ax2 - ax1
    ha = ay2 - ay1
    cxa = ax1 + 0.5 * wa
    cya = ay1 + 0.5 * ha
    cx = d0 * wa + cxa
    cy = d1 * ha + cya
    w = wa * jnp.exp(d2)
    h = ha * jnp.exp(d3)
    bx1 = cx - 0.5 * w
    by1 = cy - 0.5 * h
    bx2 = cx + 0.5 * w
    by2 = cy + 0.5 * h

    s = 1.0 / (1.0 + jnp.exp(-logit))

    off = lab * 4096.0
    ox1 = bx1 + off
    oy1 = by1 + off
    ox2 = bx2 + off
    oy2 = by2 + off
    areas = jnp.maximum(ox2 - ox1, 0.0) * jnp.maximum(oy2 - oy1, 0.0)

    cur0 = jnp.where(s > _THR, s, -1.0)

    idx = (jax.lax.broadcasted_iota(jnp.int32, (8, 128), 0) * 128
           + jax.lax.broadcasted_iota(jnp.int32, (8, 128), 1))
    lane = jax.lax.broadcasted_iota(jnp.int32, (1, 128), 1)
    rowid = jax.lax.broadcasted_iota(jnp.int32, (_OUT, 128), 0)
    rows0 = jnp.zeros((_OUT, 128), jnp.float32)

    def body(t, carry):
        cur, rows = carry
        m = jnp.max(cur)
        keep = m > 0.0
        sel = jnp.min(jnp.where(cur == m, idx, jnp.int32(1 << 30)))
        sm = idx == sel

        def pick(v):
            return jnp.sum(jnp.where(sm, v, 0.0))

        px1 = pick(bx1)
        py1 = pick(by1)
        px2 = pick(bx2)
        py2 = pick(by2)
        plab = pick(lab)
        qx1 = pick(ox1)
        qy1 = pick(oy1)
        qx2 = pick(ox2)
        qy2 = pick(oy2)
        parea = pick(areas)

        inter = (jnp.maximum(jnp.minimum(qx2, ox2) - jnp.maximum(qx1, ox1), 0.0)
                 * jnp.maximum(jnp.minimum(qy2, oy2) - jnp.maximum(qy1, oy1), 0.0))
        iou = inter / (parea + areas - inter + 1e-9)
        cur = jnp.where(iou > _IOU, -1.0, cur)

        keepf = jnp.where(keep, 1.0, 0.0)
        rowv = (jnp.where(lane == 0, px1, 0.0)
                + jnp.where(lane == 1, py1, 0.0)
                + jnp.where(lane == 2, px2, 0.0)
                + jnp.where(lane == 3, py2, 0.0)
                + jnp.where(lane == 4, m, 0.0)
                + jnp.where(lane == 5, plab, 0.0)) * keepf
        rows = jnp.where(rowid == t, rowv, rows)
        return cur, rows

    _, rows = jax.lax.fori_loop(0, _OUT, body, (cur0, rows0))
    out_ref[0] = rows


@jax.jit
def _impl(logits, regress, anchors):
    flat = jnp.concatenate([logits[lv].reshape(_B, -1) for lv in range(5)],
                           axis=1)
    flat = jnp.pad(flat, ((0, 0), (0, _NP - _N)), constant_values=-1e30)
    fx = flat.reshape(_B, _CHUNKS, _CS)

    cmax = pl.pallas_call(
        _chunkmax_body,
        grid=(_B, _CHUNKS // _CBLK),
        in_specs=[pl.BlockSpec((1, _CBLK, _CS), lambda b, i: (b, i, 0))],
        out_specs=pl.BlockSpec((1, _CBLK, 8), lambda b, i: (b, i, 0)),
        out_shape=jax.ShapeDtypeStruct((_B, _CHUNKS, 8), jnp.float32),
    )(fx)[:, :, 0]

    _, cid = jax.lax.top_k(cmax, _KC)
    cand = jnp.take_along_axis(fx, cid[:, :, None], axis=1)
    sc, p = jax.lax.top_k(cand.reshape(_B, _KC * _CS), _KC)
    cj = jnp.take_along_axis(cid, p // _CS, axis=1)
    j = cj * _CS + p % _CS

    zero = jnp.zeros_like(j)
    a = zero
    c = zero
    r0 = zero
    rs = zero
    base = 0
    off_a = 0
    off_r = 0
    for s in _SP:
        m = (j >= base) & (j < base + _A * _C * s)
        local = j - base
        ch = local // s
        pos = local % s
        k = ch // _C
        cc = ch % _C
        a = jnp.where(m, off_a + pos * _A + k, a)
        c = jnp.where(m, cc, c)
        r0 = jnp.where(m, off_r + (k * 4) * s + pos, r0)
        rs = jnp.where(m, s, rs)
        base += _A * _C * s
        off_a += _A * s
        off_r += _A * 4 * s

    rflat = jnp.concatenate([regress[lv].reshape(_B, -1) for lv in range(5)],
                            axis=1)
    d = [jnp.take_along_axis(rflat, r0 + t * rs, axis=1) for t in range(4)]

    gidx = a * _C + c
    srt = jax.lax.sort(
        (-sc, gidx, a, c, d[0], d[1], d[2], d[3]),
        dimension=1, num_keys=2)
    ms = -srt[0][:, :_K]
    a_sel = srt[2][:, :_K]
    c_sel = srt[3][:, :_K]
    d_sel = jnp.stack([srt[4][:, :_K], srt[5][:, :_K],
                       srt[6][:, :_K], srt[7][:, :_K]], axis=-1)
    anc = anchors[a_sel]

    npad = _PAD - _K

    def p(v, fill):
        return jnp.pad(v, ((0, 0), (0, npad)), constant_values=fill)

    packed = jnp.stack(
        [p(anc[..., 0], 0.0), p(anc[..., 1], 0.0),
         p(anc[..., 2], 0.0), p(anc[..., 3], 0.0),
         p(d_sel[..., 0], 0.0), p(d_sel[..., 1], 0.0),
         p(d_sel[..., 2], 0.0), p(d_sel[..., 3], 0.0),
         p(ms, -1e9), p(c_sel.astype(jnp.float32), 0.0)],
        axis=1).reshape(_B, 10, 8, 128)

    out = pl.pallas_call(
        _nms_body,
        grid=(_B,),
        in_specs=[pl.BlockSpec((1, 10, 8, 128), lambda b: (b, 0, 0, 0))],
        out_specs=pl.BlockSpec((1, _OUT, 128), lambda b: (b, 0, 0)),
        out_shape=jax.ShapeDtypeStruct((_B, _OUT, 128), jnp.float32),
    )(packed)
    return out[:, :, :6]


def kernel(logits0, logits1, logits2, logits3, logits4,
           regress0, regress1, regress2, regress3, regress4, anchors):
    return _impl([logits0, logits1, logits2, logits3, logits4],
                 [regress0, regress1, regress2, regress3, regress4], anchors)
